# Initial kernel scaffold; baseline (speedup 1.0000x reference)
import jax
import jax.numpy as jnp
from jax.experimental import pallas as pl


def kernel(x, edge_index, edge_attr, batch, W1, b1, W2, b2, W3, b3, Wl, bl):
    raise NotImplementedError("write your pallas kernel here")


# speedup vs baseline: 3.0322x; 3.0322x over previous
import functools

import jax
import jax.numpy as jnp
from jax import lax
from jax.experimental import pallas as pl
from jax.experimental.pallas import tpu as pltpu
from jax.experimental.pallas import tpu_sc as plsc

N = 50000
NP = 50048
E = 800000
EP = 819200
PAD = EP - E
NG = 64
NCLS = 5
D = 128
R = 2000
GRID = N // R
G = 128
TILES = 16
AGG_PER_TILE = EP // TILES
AGG_GROUPS = AGG_PER_TILE // G
DEG_PER_TILE = EP // 32
DEG_GROUPS = DEG_PER_TILE // G
CH = 12512
RZ = 784

f32 = jnp.float32
i32 = jnp.int32

_MESH = plsc.VectorSubcoreMesh(
    core_axis_name="c", subcore_axis_name="s", num_cores=2, num_subcores=16)


@functools.partial(
    pl.kernel,
    out_type=jax.ShapeDtypeStruct((32, NP), f32),
    mesh=_MESH,
    scratch_types=[
        pltpu.VMEM((NP,), f32),
        pltpu.VMEM((G,), i32),
        pltpu.VMEM((G,), f32),
    ],
    compiler_params=pltpu.CompilerParams(needs_layout_passes=False),
)
def _deg_call(dstr, wr, zrosnp, degp, acc, dstv, wv):
    cid = lax.axis_index("c")
    sid = lax.axis_index("s")
    wid = sid * 2 + cid
    pltpu.sync_copy(zrosnp, acc)

    def grp(g, carry):
        b = wid * DEG_PER_TILE + g * G
        pltpu.sync_copy(dstr.at[pl.ds(b, G)], dstv)
        pltpu.sync_copy(wr.at[pl.ds(b, G)], wv)

        def sub(i, c2):
            idx16 = dstv[pl.ds(i * 16, 16)]
            w16 = wv[pl.ds(i * 16, 16)]
            plsc.addupdate_scatter(acc, [idx16], w16)
            return c2

        lax.fori_loop(0, G // 16, sub, 0)
        return carry

    lax.fori_loop(0, DEG_GROUPS, grp, 0)
    pltpu.sync_copy(acc, degp.at[wid])


@functools.partial(
    pl.kernel,
    out_type=jax.ShapeDtypeStruct((N, D), f32),
    mesh=_MESH,
    scratch_types=[
        pltpu.VMEM_SHARED((CH, D), f32),
        pltpu.VMEM((G,), i32),
        pltpu.VMEM((G,), i32),
        pltpu.VMEM((G,), f32),
        pltpu.VMEM((G,), i32),
        pltpu.VMEM((G,), f32),
        pltpu.VMEM((G, D), f32),
        pltpu.SemaphoreType.DMA,
    ],
)
def _agg_call(zt, srcr, dstr, wr, zrosch, aout,
              acc, srcv, dstv, wv, relv, wv2, rows, sem):
    cid = lax.axis_index("c")
    sid = lax.axis_index("s")
    iota16 = lax.iota(i32, 16)

    def process(c):
        base = c * CH
        size_c = min(CH, N - base)
        @pl.when(sid < 15)
        def _():
            pltpu.sync_copy(zrosch, acc.at[pl.ds(sid * RZ, RZ)])

        @pl.when(sid == 15)
        def _():
            pltpu.sync_copy(zrosch.at[pl.ds(0, CH - 15 * RZ)],
                            acc.at[pl.ds(15 * RZ, CH - 15 * RZ)])

        plsc.subcore_barrier()

        def grp(g, carry):
            b = sid * AGG_PER_TILE + g * G
            pltpu.sync_copy(srcr.at[pl.ds(b, G)], srcv)
            pltpu.sync_copy(dstr.at[pl.ds(b, G)], dstv)
            pltpu.sync_copy(wr.at[pl.ds(b, G)], wv)

            def mk(i, c2):
                d16 = dstv[pl.ds(i * 16, 16)]
                w16 = wv[pl.ds(i * 16, 16)]
                m = (d16 >= base) & (d16 < base + CH)
                relv[pl.ds(i * 16, 16)] = jnp.where(m, d16 - base,
                                                    iota16 + i * 16)
                wv2[pl.ds(i * 16, 16)] = jnp.where(m, w16, f32(0.0))
                return c2

            lax.fori_loop(0, G // 16, mk, 0)
            pltpu.async_copy(zt.at[srcv], rows, sem).wait()

            def sc(i, c2):
                w16 = wv2[pl.ds(i * 16, 16)]
                for j in range(16):
                    e = i * 16 + j
                    wb = jnp.full((16,), w16[j], f32)
                    for q in range(D // 16):
                        rows[e, pl.ds(q * 16, 16)] = (
                            rows[e, pl.ds(q * 16, 16)] * wb)
                return c2

            lax.fori_loop(0, G // 16, sc, 0)
            pltpu.sync_copy(rows, acc.at[relv], add=True)
            return carry

        lax.fori_loop(0, AGG_GROUPS, grp, 0)
        plsc.subcore_barrier()
        @pl.when(sid < 15)
        def _():
            pltpu.sync_copy(acc.at[pl.ds(sid * RZ, RZ)],
                            aout.at[pl.ds(base + sid * RZ, RZ)])

        @pl.when(sid == 15)
        def _():
            pltpu.sync_copy(acc.at[pl.ds(15 * RZ, size_c - 15 * RZ)],
                            aout.at[pl.ds(base + 15 * RZ, size_c - 15 * RZ)])

        plsc.subcore_barrier()

    @pl.when(cid == 0)
    def _():
        process(0)
        process(1)

    @pl.when(cid == 1)
    def _():
        process(2)
        process(3)


def _dinv_of(dg_ref):
    deg = jnp.sum(dg_ref[...], axis=1) + 1.0
    return lax.rsqrt(deg)


def _t1_body(x_ref, w_ref, dg_ref, z_ref):
    dinv = _dinv_of(dg_ref)
    y = jnp.dot(x_ref[...], w_ref[...], preferred_element_type=f32)
    z_ref[...] = y * dinv[:, None]


def _t1(xp, W1p, degp):
    return pl.pallas_call(
        _t1_body,
        grid=(GRID,),
        in_specs=[
            pl.BlockSpec((R, 8), lambda i: (i, 0)),
            pl.BlockSpec((8, D), lambda i: (0, 0)),
            pl.BlockSpec((R, 32), lambda i: (i, 0)),
        ],
        out_specs=pl.BlockSpec((R, D), lambda i: (i, 0)),
        out_shape=jax.ShapeDtypeStruct((N, D), f32),
    )(xp, W1p, degp)


def _mid_body(a_ref, z_ref, dg_ref, b_ref, w_ref, o_ref):
    dinv = _dinv_of(dg_ref)
    h = jnp.maximum(dinv[:, None] * (a_ref[...] + z_ref[...]) + b_ref[...],
                    0.0)
    y = jnp.dot(h, w_ref[...], preferred_element_type=f32)
    o_ref[...] = y * dinv[:, None]


def _mid(a, z, b, W, degp):
    return pl.pallas_call(
        _mid_body,
        grid=(GRID,),
        in_specs=[
            pl.BlockSpec((R, D), lambda i: (i, 0)),
            pl.BlockSpec((R, D), lambda i: (i, 0)),
            pl.BlockSpec((R, 32), lambda i: (i, 0)),
            pl.BlockSpec((1, D), lambda i: (0, 0)),
            pl.BlockSpec((D, D), lambda i: (0, 0)),
        ],
        out_specs=pl.BlockSpec((R, D), lambda i: (i, 0)),
        out_shape=jax.ShapeDtypeStruct((N, D), f32),
    )(a, z, degp, b.reshape(1, D), W)


def _t4_body(a_ref, z_ref, dg_ref, b3_ref, bt_ref, wl_ref, bl_ref, o_ref,
             acc, cnt):
    i = pl.program_id(0)
    dinv = _dinv_of(dg_ref)
    h = dinv[:, None] * (a_ref[...] + z_ref[...]) + b3_ref[...]
    bb = bt_ref[0, 0, :]
    oh = (bb[None, :] == lax.broadcasted_iota(i32, (NG, R), 0)).astype(f32)

    @pl.when(i == 0)
    def _():
        acc[...] = jnp.zeros((NG, D), f32)
        cnt[...] = jnp.zeros((NG, D), f32)

    acc[...] += jnp.dot(oh, h, preferred_element_type=f32)
    cnt[...] += jnp.broadcast_to(jnp.sum(oh, axis=1)[:, None], (NG, D))

    @pl.when(i == GRID - 1)
    def _():
        pooled = acc[...] / jnp.maximum(cnt[...], 1.0)
        o_ref[...] = jnp.dot(pooled, wl_ref[...],
                             preferred_element_type=f32) + bl_ref[...]


def _t4(a, z, degp, b3, bt, Wlp, blp):
    return pl.pallas_call(
        _t4_body,
        grid=(GRID,),
        in_specs=[
            pl.BlockSpec((R, D), lambda i: (i, 0)),
            pl.BlockSpec((R, D), lambda i: (i, 0)),
            pl.BlockSpec((R, 32), lambda i: (i, 0)),
            pl.BlockSpec((1, D), lambda i: (0, 0)),
            pl.BlockSpec((1, 1, R), lambda i: (i, 0, 0)),
            pl.BlockSpec((D, 8), lambda i: (0, 0)),
            pl.BlockSpec((1, 8), lambda i: (0, 0)),
        ],
        out_specs=pl.BlockSpec((NG, 8), lambda i: (0, 0)),
        out_shape=jax.ShapeDtypeStruct((NG, 8), f32),
        scratch_shapes=[
            pltpu.VMEM((NG, D), f32),
            pltpu.VMEM((NG, D), f32),
        ],
    )(a, z, degp, b3.reshape(1, D), bt, Wlp, blp)


def kernel(x, edge_index, edge_attr, batch, W1, b1, W2, b2, W3, b3, Wl, bl):
    src = edge_index[0].astype(i32)
    dst = edge_index[1].astype(i32)
    padidx = (jnp.arange(PAD, dtype=i32) * 2503) % N
    srcp = jnp.concatenate([src, padidx])
    dstp = jnp.concatenate([dst, padidx])
    wp = jnp.concatenate([edge_attr.astype(f32), jnp.zeros((PAD,), f32)])

    degp = _deg_call(dstp, wp, jnp.zeros((NP,), f32))

    xp = jnp.pad(x, ((0, 0), (0, 1)))
    W1p = jnp.pad(W1, ((0, 1), (0, 0)))
    zrosch = jnp.zeros((RZ, D), f32)
    dg = degp[:, :N].T

    z1 = _t1(xp, W1p, dg)
    a1 = _agg_call(z1, srcp, dstp, wp, zrosch)
    z2 = _mid(a1, z1, b1, W2, dg)
    a2 = _agg_call(z2, srcp, dstp, wp, zrosch)
    z3 = _mid(a2, z2, b2, W3, dg)
    a3 = _agg_call(z3, srcp, dstp, wp, zrosch)

    bt = batch.astype(i32).reshape(GRID, 1, R)
    Wlp = jnp.pad(Wl, ((0, 0), (0, 8 - NCLS)))
    blp = jnp.pad(bl, (0, 8 - NCLS)).reshape(1, 8)
    out = _t4(a3, z3, dg, b3, bt, Wlp, blp)
    return out[:, :NCLS]


# speedup vs baseline: 4.0847x; 1.3471x over previous
import functools

import jax
import jax.numpy as jnp
from jax import lax
from jax.experimental import pallas as pl
from jax.experimental.pallas import tpu as pltpu
from jax.experimental.pallas import tpu_sc as plsc

N = 50000
NP = 50048
E = 800000
EP = 819200
PAD = EP - E
NG = 64
NCLS = 5
D = 128
R = 2000
GRID = N // R
G = 80
TILES = 16
AGG_PER_TILE = EP // TILES
AGG_GROUPS = AGG_PER_TILE // G
DEG_PER_TILE = EP // 32
DEG_GROUPS = DEG_PER_TILE // G
CH = 12512
RZ = 784

f32 = jnp.float32
i32 = jnp.int32

_MESH = plsc.VectorSubcoreMesh(
    core_axis_name="c", subcore_axis_name="s", num_cores=2, num_subcores=16)


@functools.partial(
    pl.kernel,
    out_type=jax.ShapeDtypeStruct((32, NP), f32),
    mesh=_MESH,
    scratch_types=[
        pltpu.VMEM((NP,), f32),
        pltpu.VMEM((G,), i32),
        pltpu.VMEM((G,), f32),
    ],
    compiler_params=pltpu.CompilerParams(needs_layout_passes=False),
)
def _deg_call(dstr, wr, zrosnp, degp, acc, dstv, wv):
    cid = lax.axis_index("c")
    sid = lax.axis_index("s")
    wid = sid * 2 + cid
    pltpu.sync_copy(zrosnp, acc)

    def grp(g, carry):
        b = wid * DEG_PER_TILE + g * G
        pltpu.sync_copy(dstr.at[pl.ds(b, G)], dstv)
        pltpu.sync_copy(wr.at[pl.ds(b, G)], wv)

        def sub(i, c2):
            idx16 = dstv[pl.ds(i * 16, 16)]
            w16 = wv[pl.ds(i * 16, 16)]
            plsc.addupdate_scatter(acc, [idx16], w16)
            return c2

        lax.fori_loop(0, G // 16, sub, 0)
        return carry

    lax.fori_loop(0, DEG_GROUPS, grp, 0)
    pltpu.sync_copy(acc, degp.at[wid])


@functools.partial(
    pl.kernel,
    out_type=jax.ShapeDtypeStruct((N, D), f32),
    mesh=_MESH,
    scratch_types=[
        pltpu.VMEM_SHARED((CH, D), f32),
        pltpu.VMEM((3, G), i32),
        pltpu.VMEM((3, G), i32),
        pltpu.VMEM((G,), i32),
        pltpu.VMEM((G,), i32),
        pltpu.VMEM((G,), i32),
        pltpu.VMEM((G, D), f32),
        pltpu.VMEM((G, D), f32),
        pltpu.SemaphoreType.DMA,
        pltpu.SemaphoreType.DMA,
    ],
    compiler_params=pltpu.CompilerParams(needs_layout_passes=False),
)
def _agg_call(zt, srcr, edr, zrosch, aout,
              acc, eda, edb, srcva, srcvb, relv, rowsa, rowsb, sg0, sg1):
    cid = lax.axis_index("c")
    sid = lax.axis_index("s")
    iota16 = lax.iota(i32, 16)
    eds = (eda, edb)
    srcvs = (srcva, srcvb)
    rowss = (rowsa, rowsb)
    sgs = (sg0, sg1)

    def process(c):
        base = c * CH
        size_c = min(CH, N - base)
        @pl.when(sid < 15)
        def _():
            pltpu.sync_copy(zrosch, acc.at[pl.ds(sid * RZ, RZ)])

        @pl.when(sid == 15)
        def _():
            pltpu.sync_copy(zrosch.at[pl.ds(0, CH - 15 * RZ)],
                            acc.at[pl.ds(15 * RZ, CH - 15 * RZ)])

        plsc.subcore_barrier()

        gbase = sid * AGG_GROUPS
        ebase = sid * AGG_PER_TILE
        pltpu.sync_copy(edr.at[gbase], eda)
        pltpu.sync_copy(srcr.at[pl.ds(ebase, G)], srcva)
        pltpu.async_copy(zt.at[srcva], rowsa, sgs[0])

        def grp2(k, carry):
            for b in range(2):
                nb = 1 - b
                g = k * 2 + b

                @pl.when(g + 1 < AGG_GROUPS)
                def _():
                    pltpu.sync_copy(edr.at[gbase + g + 1], eds[nb])
                    pltpu.sync_copy(srcr.at[pl.ds(ebase + (g + 1) * G, G)],
                                    srcvs[nb])
                    pltpu.async_copy(zt.at[srcvs[nb]], rowss[nb], sgs[nb])

                pltpu.make_async_copy(zt.at[srcvs[b]], rowss[b],
                                      sgs[b]).wait()
                rows = rowss[b]
                ed = eds[b]

                def sc(i, c2):
                    d16 = ed[1, pl.ds(i * 16, 16)]
                    w16 = plsc.bitcast(ed[2, pl.ds(i * 16, 16)], f32)
                    m = (d16 >= base) & (d16 < base + CH)
                    relv[pl.ds(i * 16, 16)] = jnp.where(m, d16 - base,
                                                        iota16 + i * 16)
                    w16m = jnp.where(m, w16, f32(0.0))
                    for j in range(16):
                        e = i * 16 + j
                        wb = jnp.full((16,), w16m[j], f32)
                        for q in range(D // 16):
                            rows[e, pl.ds(q * 16, 16)] = (
                                rows[e, pl.ds(q * 16, 16)] * wb)
                    return c2

                lax.fori_loop(0, G // 16, sc, 0)
                pltpu.sync_copy(rows, acc.at[relv], add=True)
            return carry

        lax.fori_loop(0, AGG_GROUPS // 2, grp2, 0)
        plsc.subcore_barrier()
        @pl.when(sid < 15)
        def _():
            pltpu.sync_copy(acc.at[pl.ds(sid * RZ, RZ)],
                            aout.at[pl.ds(base + sid * RZ, RZ)])

        @pl.when(sid == 15)
        def _():
            pltpu.sync_copy(acc.at[pl.ds(15 * RZ, size_c - 15 * RZ)],
                            aout.at[pl.ds(base + 15 * RZ, size_c - 15 * RZ)])

        plsc.subcore_barrier()

    @pl.when(cid == 0)
    def _():
        process(0)
        process(1)

    @pl.when(cid == 1)
    def _():
        process(2)
        process(3)


def _dinv_of(dg_ref):
    deg = jnp.sum(dg_ref[...], axis=1) + 1.0
    return lax.rsqrt(deg)


def _t1_body(x_ref, w_ref, dg_ref, z_ref):
    dinv = _dinv_of(dg_ref)
    y = jnp.dot(x_ref[...], w_ref[...], preferred_element_type=f32)
    z_ref[...] = y * dinv[:, None]


def _t1(xp, W1p, degp):
    return pl.pallas_call(
        _t1_body,
        grid=(GRID,),
        in_specs=[
            pl.BlockSpec((R, 8), lambda i: (i, 0)),
            pl.BlockSpec((8, D), lambda i: (0, 0)),
            pl.BlockSpec((R, 32), lambda i: (i, 0)),
        ],
        out_specs=pl.BlockSpec((R, D), lambda i: (i, 0)),
        out_shape=jax.ShapeDtypeStruct((N, D), f32),
    )(xp, W1p, degp)


def _mid_body(a_ref, z_ref, dg_ref, b_ref, w_ref, o_ref):
    dinv = _dinv_of(dg_ref)
    h = jnp.maximum(dinv[:, None] * (a_ref[...] + z_ref[...]) + b_ref[...],
                    0.0)
    y = jnp.dot(h, w_ref[...], preferred_element_type=f32)
    o_ref[...] = y * dinv[:, None]


def _mid(a, z, b, W, degp):
    return pl.pallas_call(
        _mid_body,
        grid=(GRID,),
        in_specs=[
            pl.BlockSpec((R, D), lambda i: (i, 0)),
            pl.BlockSpec((R, D), lambda i: (i, 0)),
            pl.BlockSpec((R, 32), lambda i: (i, 0)),
            pl.BlockSpec((1, D), lambda i: (0, 0)),
            pl.BlockSpec((D, D), lambda i: (0, 0)),
        ],
        out_specs=pl.BlockSpec((R, D), lambda i: (i, 0)),
        out_shape=jax.ShapeDtypeStruct((N, D), f32),
    )(a, z, degp, b.reshape(1, D), W)


def _t4_body(a_ref, z_ref, dg_ref, b3_ref, bt_ref, wl_ref, bl_ref, o_ref,
             acc, cnt):
    i = pl.program_id(0)
    dinv = _dinv_of(dg_ref)
    h = dinv[:, None] * (a_ref[...] + z_ref[...]) + b3_ref[...]
    bb = bt_ref[0, 0, :]
    oh = (bb[None, :] == lax.broadcasted_iota(i32, (NG, R), 0)).astype(f32)

    @pl.when(i == 0)
    def _():
        acc[...] = jnp.zeros((NG, D), f32)
        cnt[...] = jnp.zeros((NG, D), f32)

    acc[...] += jnp.dot(oh, h, preferred_element_type=f32)
    cnt[...] += jnp.broadcast_to(jnp.sum(oh, axis=1)[:, None], (NG, D))

    @pl.when(i == GRID - 1)
    def _():
        pooled = acc[...] / jnp.maximum(cnt[...], 1.0)
        o_ref[...] = jnp.dot(pooled, wl_ref[...],
                             preferred_element_type=f32) + bl_ref[...]


def _t4(a, z, degp, b3, bt, Wlp, blp):
    return pl.pallas_call(
        _t4_body,
        grid=(GRID,),
        in_specs=[
            pl.BlockSpec((R, D), lambda i: (i, 0)),
            pl.BlockSpec((R, D), lambda i: (i, 0)),
            pl.BlockSpec((R, 32), lambda i: (i, 0)),
            pl.BlockSpec((1, D), lambda i: (0, 0)),
            pl.BlockSpec((1, 1, R), lambda i: (i, 0, 0)),
            pl.BlockSpec((D, 8), lambda i: (0, 0)),
            pl.BlockSpec((1, 8), lambda i: (0, 0)),
        ],
        out_specs=pl.BlockSpec((NG, 8), lambda i: (0, 0)),
        out_shape=jax.ShapeDtypeStruct((NG, 8), f32),
        scratch_shapes=[
            pltpu.VMEM((NG, D), f32),
            pltpu.VMEM((NG, D), f32),
        ],
    )(a, z, degp, b3.reshape(1, D), bt, Wlp, blp)


def kernel(x, edge_index, edge_attr, batch, W1, b1, W2, b2, W3, b3, Wl, bl):
    src = edge_index[0].astype(i32)
    dst = edge_index[1].astype(i32)
    padidx = (jnp.arange(PAD, dtype=i32) * 2503) % N
    srcp = jnp.concatenate([src, padidx])
    dstp = jnp.concatenate([dst, padidx])
    wp = jnp.concatenate([edge_attr.astype(f32), jnp.zeros((PAD,), f32)])

    degp = _deg_call(dstp, wp, jnp.zeros((NP,), f32))

    xp = jnp.pad(x, ((0, 0), (0, 1)))
    W1p = jnp.pad(W1, ((0, 1), (0, 0)))
    zrosch = jnp.zeros((RZ, D), f32)
    dg = degp[:, :N].T
    wbits = lax.bitcast_convert_type(wp, i32)
    ed3 = jnp.stack([srcp.reshape(EP // G, G), dstp.reshape(EP // G, G),
                     wbits.reshape(EP // G, G)], axis=1)

    z1 = _t1(xp, W1p, dg)
    a1 = _agg_call(z1, srcp, ed3, zrosch)
    z2 = _mid(a1, z1, b1, W2, dg)
    a2 = _agg_call(z2, srcp, ed3, zrosch)
    z3 = _mid(a2, z2, b2, W3, dg)
    a3 = _agg_call(z3, srcp, ed3, zrosch)

    bt = batch.astype(i32).reshape(GRID, 1, R)
    Wlp = jnp.pad(Wl, ((0, 0), (0, 8 - NCLS)))
    blp = jnp.pad(bl, (0, 8 - NCLS)).reshape(1, 8)
    out = _t4(a3, z3, dg, b3, bt, Wlp, blp)
    return out[:, :NCLS]


# speedup vs baseline: 6.4706x; 1.5841x over previous
import functools

import jax
import jax.numpy as jnp
from jax import lax
from jax.experimental import pallas as pl
from jax.experimental.pallas import tpu as pltpu
from jax.experimental.pallas import tpu_sc as plsc

N = 50000
NP = 50048
E = 800000
EP = 819200
PAD = EP - E
NG = 64
NCLS = 5
D = 128
R = 2000
GRID = N // R
G = 80
TILES = 16
AGG_PER_TILE = EP // TILES
AGG_GROUPS = AGG_PER_TILE // G
DEG_PER_TILE = EP // 32
DEG_GROUPS = DEG_PER_TILE // G
CH = 12512
RZ = 784

f32 = jnp.float32
i32 = jnp.int32

_MESH = plsc.VectorSubcoreMesh(
    core_axis_name="c", subcore_axis_name="s", num_cores=2, num_subcores=16)


@functools.partial(
    pl.kernel,
    out_type=jax.ShapeDtypeStruct((32, NP), f32),
    mesh=_MESH,
    scratch_types=[
        pltpu.VMEM((NP,), f32),
        pltpu.VMEM((G,), i32),
        pltpu.VMEM((G,), f32),
    ],
    compiler_params=pltpu.CompilerParams(needs_layout_passes=False),
)
def _deg_call(dstr, wr, zrosnp, degp, acc, dstv, wv):
    cid = lax.axis_index("c")
    sid = lax.axis_index("s")
    wid = sid * 2 + cid
    pltpu.sync_copy(zrosnp, acc)

    def grp(g, carry):
        b = wid * DEG_PER_TILE + g * G
        pltpu.sync_copy(dstr.at[pl.ds(b, G)], dstv)
        pltpu.sync_copy(wr.at[pl.ds(b, G)], wv)

        def sub(i, c2):
            idx16 = dstv[pl.ds(i * 16, 16)]
            w16 = wv[pl.ds(i * 16, 16)]
            plsc.addupdate_scatter(acc, [idx16], w16)
            return c2

        lax.fori_loop(0, G // 16, sub, 0)
        return carry

    lax.fori_loop(0, DEG_GROUPS, grp, 0)
    pltpu.sync_copy(acc, degp.at[wid])


@functools.partial(
    pl.kernel,
    out_type=jax.ShapeDtypeStruct((N, D), f32),
    mesh=_MESH,
    scratch_types=[
        pltpu.VMEM_SHARED((CH, D), f32),
        pltpu.VMEM((3, G), i32),
        pltpu.VMEM((3, G), i32),
        pltpu.VMEM((G,), i32),
        pltpu.VMEM((G,), i32),
        pltpu.VMEM((G,), i32),
        pltpu.VMEM((G,), i32),
        pltpu.VMEM((G, D), f32),
        pltpu.VMEM((G, D), f32),
        pltpu.SemaphoreType.DMA,
        pltpu.SemaphoreType.DMA,
        pltpu.SemaphoreType.DMA,
        pltpu.SemaphoreType.DMA,
        pltpu.SemaphoreType.DMA,
        pltpu.SemaphoreType.DMA,
    ],
    compiler_params=pltpu.CompilerParams(needs_layout_passes=False),
)
def _agg_call(zt, srcr, edr, zrosch, aout,
              acc, eda, edb, srcva, srcvb, relva, relvb, rowsa, rowsb,
              sg0, sg1, sl0, sl1, ss0, ss1):
    cid = lax.axis_index("c")
    sid = lax.axis_index("s")
    iota16 = lax.iota(i32, 16)
    eds = (eda, edb)
    srcvs = (srcva, srcvb)
    relvs = (relva, relvb)
    rowss = (rowsa, rowsb)
    sgs = (sg0, sg1)
    sls = (sl0, sl1)
    sss = (ss0, ss1)

    def process(c):
        base = c * CH
        size_c = min(CH, N - base)
        @pl.when(sid < 15)
        def _():
            pltpu.sync_copy(zrosch, acc.at[pl.ds(sid * RZ, RZ)])

        @pl.when(sid == 15)
        def _():
            pltpu.sync_copy(zrosch.at[pl.ds(0, CH - 15 * RZ)],
                            acc.at[pl.ds(15 * RZ, CH - 15 * RZ)])

        plsc.subcore_barrier()

        gbase = sid * AGG_GROUPS
        ebase = sid * AGG_PER_TILE
        pltpu.sync_copy(edr.at[gbase], eda)
        pltpu.sync_copy(srcr.at[pl.ds(ebase, G)], srcva)
        pltpu.async_copy(zt.at[srcva], rowsa, sgs[0])
        pltpu.async_copy(edr.at[gbase + 1], edb, sls[1])
        pltpu.async_copy(srcr.at[pl.ds(ebase + G, G)], srcvb, sls[1])

        def grp2(k, carry):
            for b in range(2):
                nb = 1 - b
                g = k * 2 + b
                ed = eds[b]
                rows = rowss[b]
                relv = relvs[b]

                @pl.when(g > 0)
                def _():
                    pltpu.make_async_copy(rowss[nb], acc.at[relvs[nb]],
                                          sss[nb]).wait()

                @pl.when(g + 1 < AGG_GROUPS)
                def _():
                    pltpu.make_async_copy(edr.at[gbase + g + 1], eds[nb],
                                          sls[nb]).wait()
                    pltpu.make_async_copy(
                        srcr.at[pl.ds(ebase + (g + 1) * G, G)],
                        srcvs[nb], sls[nb]).wait()
                    pltpu.async_copy(zt.at[srcvs[nb]], rowss[nb], sgs[nb])

                pltpu.make_async_copy(zt.at[srcvs[b]], rows, sgs[b]).wait()

                def sc(i, c2):
                    d16 = ed[1, pl.ds(i * 16, 16)]
                    w16 = plsc.bitcast(ed[2, pl.ds(i * 16, 16)], f32)
                    m = (d16 >= base) & (d16 < base + CH)
                    relv[pl.ds(i * 16, 16)] = jnp.where(m, d16 - base,
                                                        iota16 + i * 16)
                    w16m = jnp.where(m, w16, f32(0.0))
                    for j in range(16):
                        e = i * 16 + j
                        wb = jnp.full((16,), w16m[j], f32)
                        for q in range(D // 16):
                            rows[e, pl.ds(q * 16, 16)] = (
                                rows[e, pl.ds(q * 16, 16)] * wb)
                    return c2

                lax.fori_loop(0, G // 16, sc, 0)

                @pl.when(g + 1 < AGG_GROUPS)
                def _():
                    pltpu.async_copy(rows, acc.at[relv], sss[b], add=True)

                @pl.when(g + 1 == AGG_GROUPS)
                def _():
                    pltpu.sync_copy(rows, acc.at[relv], add=True)

                @pl.when(g + 2 < AGG_GROUPS)
                def _():
                    pltpu.async_copy(edr.at[gbase + g + 2], ed, sls[b])
                    pltpu.async_copy(
                        srcr.at[pl.ds(ebase + (g + 2) * G, G)],
                        srcvs[b], sls[b])
            return carry

        lax.fori_loop(0, AGG_GROUPS // 2, grp2, 0)
        plsc.subcore_barrier()
        @pl.when(sid < 15)
        def _():
            pltpu.sync_copy(acc.at[pl.ds(sid * RZ, RZ)],
                            aout.at[pl.ds(base + sid * RZ, RZ)])

        @pl.when(sid == 15)
        def _():
            pltpu.sync_copy(acc.at[pl.ds(15 * RZ, size_c - 15 * RZ)],
                            aout.at[pl.ds(base + 15 * RZ, size_c - 15 * RZ)])

        plsc.subcore_barrier()

    @pl.when(cid == 0)
    def _():
        process(0)
        process(1)

    @pl.when(cid == 1)
    def _():
        process(2)
        process(3)


def _dinv_of(dg_ref):
    deg = jnp.sum(dg_ref[...], axis=1) + 1.0
    return lax.rsqrt(deg)


def _t1_body(x_ref, w_ref, dg_ref, z_ref):
    dinv = _dinv_of(dg_ref)
    y = jnp.dot(x_ref[...], w_ref[...], preferred_element_type=f32)
    z_ref[...] = y * dinv[:, None]


def _t1(xp, W1p, degp):
    return pl.pallas_call(
        _t1_body,
        grid=(GRID,),
        in_specs=[
            pl.BlockSpec((R, 8), lambda i: (i, 0)),
            pl.BlockSpec((8, D), lambda i: (0, 0)),
            pl.BlockSpec((R, 32), lambda i: (i, 0)),
        ],
        out_specs=pl.BlockSpec((R, D), lambda i: (i, 0)),
        out_shape=jax.ShapeDtypeStruct((N, D), f32),
    )(xp, W1p, degp)


def _mid_body(a_ref, z_ref, dg_ref, b_ref, w_ref, o_ref):
    dinv = _dinv_of(dg_ref)
    h = jnp.maximum(dinv[:, None] * (a_ref[...] + z_ref[...]) + b_ref[...],
                    0.0)
    y = jnp.dot(h, w_ref[...], preferred_element_type=f32)
    o_ref[...] = y * dinv[:, None]


def _mid(a, z, b, W, degp):
    return pl.pallas_call(
        _mid_body,
        grid=(GRID,),
        in_specs=[
            pl.BlockSpec((R, D), lambda i: (i, 0)),
            pl.BlockSpec((R, D), lambda i: (i, 0)),
            pl.BlockSpec((R, 32), lambda i: (i, 0)),
            pl.BlockSpec((1, D), lambda i: (0, 0)),
            pl.BlockSpec((D, D), lambda i: (0, 0)),
        ],
        out_specs=pl.BlockSpec((R, D), lambda i: (i, 0)),
        out_shape=jax.ShapeDtypeStruct((N, D), f32),
    )(a, z, degp, b.reshape(1, D), W)


def _t4_body(a_ref, z_ref, dg_ref, b3_ref, bt_ref, wl_ref, bl_ref, o_ref,
             acc, cnt):
    i = pl.program_id(0)
    dinv = _dinv_of(dg_ref)
    h = dinv[:, None] * (a_ref[...] + z_ref[...]) + b3_ref[...]
    bb = bt_ref[0, 0, :]
    oh = (bb[None, :] == lax.broadcasted_iota(i32, (NG, R), 0)).astype(f32)

    @pl.when(i == 0)
    def _():
        acc[...] = jnp.zeros((NG, D), f32)
        cnt[...] = jnp.zeros((NG, D), f32)

    acc[...] += jnp.dot(oh, h, preferred_element_type=f32)
    cnt[...] += jnp.broadcast_to(jnp.sum(oh, axis=1)[:, None], (NG, D))

    @pl.when(i == GRID - 1)
    def _():
        pooled = acc[...] / jnp.maximum(cnt[...], 1.0)
        o_ref[...] = jnp.dot(pooled, wl_ref[...],
                             preferred_element_type=f32) + bl_ref[...]


def _t4(a, z, degp, b3, bt, Wlp, blp):
    return pl.pallas_call(
        _t4_body,
        grid=(GRID,),
        in_specs=[
            pl.BlockSpec((R, D), lambda i: (i, 0)),
            pl.BlockSpec((R, D), lambda i: (i, 0)),
            pl.BlockSpec((R, 32), lambda i: (i, 0)),
            pl.BlockSpec((1, D), lambda i: (0, 0)),
            pl.BlockSpec((1, 1, R), lambda i: (i, 0, 0)),
            pl.BlockSpec((D, 8), lambda i: (0, 0)),
            pl.BlockSpec((1, 8), lambda i: (0, 0)),
        ],
        out_specs=pl.BlockSpec((NG, 8), lambda i: (0, 0)),
        out_shape=jax.ShapeDtypeStruct((NG, 8), f32),
        scratch_shapes=[
            pltpu.VMEM((NG, D), f32),
            pltpu.VMEM((NG, D), f32),
        ],
    )(a, z, degp, b3.reshape(1, D), bt, Wlp, blp)


def kernel(x, edge_index, edge_attr, batch, W1, b1, W2, b2, W3, b3, Wl, bl):
    src = edge_index[0].astype(i32)
    dst = edge_index[1].astype(i32)
    padidx = (jnp.arange(PAD, dtype=i32) * 2503) % N
    srcp = jnp.concatenate([src, padidx])
    dstp = jnp.concatenate([dst, padidx])
    wp = jnp.concatenate([edge_attr.astype(f32), jnp.zeros((PAD,), f32)])

    degp = _deg_call(dstp, wp, jnp.zeros((NP,), f32))

    xp = jnp.pad(x, ((0, 0), (0, 1)))
    W1p = jnp.pad(W1, ((0, 1), (0, 0)))
    zrosch = jnp.zeros((RZ, D), f32)
    dg = degp[:, :N].T
    wbits = lax.bitcast_convert_type(wp, i32)
    ed3 = jnp.stack([srcp.reshape(EP // G, G), dstp.reshape(EP // G, G),
                     wbits.reshape(EP // G, G)], axis=1)

    z1 = _t1(xp, W1p, dg)
    a1 = _agg_call(z1, srcp, ed3, zrosch)
    z2 = _mid(a1, z1, b1, W2, dg)
    a2 = _agg_call(z2, srcp, ed3, zrosch)
    z3 = _mid(a2, z2, b2, W3, dg)
    a3 = _agg_call(z3, srcp, ed3, zrosch)

    bt = batch.astype(i32).reshape(GRID, 1, R)
    Wlp = jnp.pad(Wl, ((0, 0), (0, 8 - NCLS)))
    blp = jnp.pad(bl, (0, 8 - NCLS)).reshape(1, 8)
    out = _t4(a3, z3, dg, b3, bt, Wlp, blp)
    return out[:, :NCLS]
